# Initial kernel scaffold; baseline (speedup 1.0000x reference)
import jax
import jax.numpy as jnp
from jax.experimental import pallas as pl


def kernel(h, batch, edge_index, h_edge_attr, W1, b1, ln1_g, ln1_b, W2, b2, ln2_g, ln2_b):
    raise NotImplementedError("write your pallas kernel here")


# speedup vs baseline: 5.0727x; 5.0727x over previous
import functools
import math

import jax
import jax.numpy as jnp
from jax import lax
from jax.experimental import pallas as pl
from jax.experimental.pallas import tpu as pltpu
from jax.experimental.pallas import tpu_sc as plsc

N = 10000
E = 320000
D = 128
L = 3

NC = 2
NS = 16
TILES = NC * NS
EPT = E // TILES
CH = 80
NCHUNK = EPT // CH
IB = 25
NG = NCHUNK // IB
ACC_R = 10240
RPS = ACC_R // NS
WPS = 640
ZR = 32

_mesh = plsc.VectorSubcoreMesh(
    core_axis_name="c", subcore_axis_name="s", num_cores=NC, num_subcores=NS)

_agg_out_type = jax.ShapeDtypeStruct((NC, N, D), jnp.float32)


@functools.partial(
    pl.kernel,
    out_type=_agg_out_type,
    mesh=_mesh,
    scratch_types=[
        pltpu.VMEM((IB, CH), jnp.int32),
        pltpu.VMEM((IB, CH), jnp.int32),
        pltpu.VMEM((CH, D), jnp.float32),
        pltpu.VMEM((CH, D), jnp.float32),
        pltpu.VMEM((ZR, D), jnp.float32),
        pltpu.SemaphoreType.DMA,
        pltpu.SemaphoreType.DMA,
        pltpu.VMEM_SHARED((ACC_R, D), jnp.float32),
    ],
)
def _sc_agg(h_hbm, ea_hbm, src_hbm, dst_hbm, out_hbm,
            src_v, dst_v, hrows, earows, zbuf, sem1, sem2, acc):
    c = lax.axis_index("c")
    s = lax.axis_index("s")

    def zstore(t, carry):
        i = t // (D // 16)
        k = t % (D // 16)
        zbuf[i, pl.ds(k * 16, 16)] = jnp.zeros((16,), jnp.float32)
        return carry
    lax.fori_loop(0, ZR * (D // 16), zstore, 0)
    for kk in range(RPS // ZR):
        pltpu.sync_copy(zbuf, acc.at[pl.ds(s * RPS + kk * ZR, ZR)])

    t = c * NS + s
    plsc.subcore_barrier()
    base = t * EPT

    def group(g, carry):
        pltpu.sync_copy(src_hbm.at[t, g], src_v)
        pltpu.sync_copy(dst_hbm.at[t, g], dst_v)

        def chunk(j, carry2):
            cp1 = pltpu.async_copy(h_hbm.at[src_v.at[j]], hrows, sem1)
            cp2 = pltpu.async_copy(
                ea_hbm.at[pl.ds(base + (g * IB + j) * CH, CH)], earows, sem2)
            cp1.wait()
            cp2.wait()
            pltpu.sync_copy(hrows, acc.at[dst_v.at[j]], add=True)
            pltpu.sync_copy(earows, acc.at[dst_v.at[j]], add=True)
            return carry2
        lax.fori_loop(0, IB, chunk, 0)
        return carry
    lax.fori_loop(0, NG, group, 0)
    plsc.subcore_barrier()
    w = jnp.where(s == NS - 1, N - WPS, s * WPS)
    pltpu.sync_copy(acc.at[pl.ds(w, WPS)], out_hbm.at[c, pl.ds(w, WPS)])


def _gelu(x):
    return 0.5 * x * (1.0 + lax.erf(x * (1.0 / math.sqrt(2.0))))


def _ln(x, g, b):
    mu = jnp.mean(x, axis=-1, keepdims=True)
    var = jnp.mean((x - mu) ** 2, axis=-1, keepdims=True)
    return (x - mu) * lax.rsqrt(var + 1e-5) * g + b


def _mlp_body(a, h_ref, w1, b1, g1, bb1, w2, b2, g2, bb2, out_ref):
    h = h_ref[...]
    x = a.at[0][...] + a.at[1][...] + h
    u = jnp.dot(x, w1[...], preferred_element_type=jnp.float32) + b1[...]
    u = _gelu(_ln(u, g1[...], bb1[...]))
    v = jnp.dot(u, w2[...], preferred_element_type=jnp.float32) + b2[...]
    v = _ln(v, g2[...], bb2[...])
    out_ref[...] = _gelu(v + h)


_ROWS = 1000


def _tc_mlp(a, h, w1, b1, g1, bb1, w2, b2, g2, bb2):
    aspec = pl.BlockSpec((NC, _ROWS, D), lambda i: (0, i, 0))
    big = pl.BlockSpec((_ROWS, D), lambda i: (i, 0))
    wspec = pl.BlockSpec((D, D), lambda i: (0, 0))
    vspec = pl.BlockSpec((1, D), lambda i: (0, 0))
    return pl.pallas_call(
        _mlp_body,
        grid=(N // _ROWS,),
        in_specs=[aspec, big,
                  wspec, vspec, vspec, vspec,
                  wspec, vspec, vspec, vspec],
        out_specs=big,
        out_shape=jax.ShapeDtypeStruct((N, D), jnp.float32),
    )(a, h, w1, b1, g1, bb1, w2, b2, g2, bb2)


def kernel(h, batch, edge_index, h_edge_attr,
           W1, b1, ln1_g, ln1_b, W2, b2, ln2_g, ln2_b):
    del batch
    src = edge_index[0].astype(jnp.int32).reshape(TILES, NG, IB, CH)
    dst = edge_index[1].astype(jnp.int32).reshape(TILES, NG, IB, CH)

    def body(x, ws):
        w1, bb1v, g1, bv1, w2, bb2v, g2, bv2 = ws
        a = _sc_agg(x, h_edge_attr, src, dst)
        x = _tc_mlp(a, x, w1, bb1v, g1, bv1, w2, bb2v, g2, bv2)
        return x, None

    ws = (W1, b1.reshape(L, 1, D), ln1_g.reshape(L, 1, D),
          ln1_b.reshape(L, 1, D), W2, b2.reshape(L, 1, D),
          ln2_g.reshape(L, 1, D), ln2_b.reshape(L, 1, D))
    x, _ = lax.scan(body, h, ws)
    return x


# speedup vs baseline: 5.5187x; 1.0879x over previous
import functools
import math

import jax
import jax.numpy as jnp
from jax import lax
from jax.experimental import pallas as pl
from jax.experimental.pallas import tpu as pltpu
from jax.experimental.pallas import tpu_sc as plsc

N = 10000
E = 320000
D = 128
L = 3

NC = 2
NS = 16
TILES = NC * NS
EPT = E // TILES
CH = 40
NCHUNK = EPT // CH
IB = 25
NG = NCHUNK // IB
ACC_R = 10240
RPS = ACC_R // NS
WPS = 640
ZR = 16

_mesh = plsc.VectorSubcoreMesh(
    core_axis_name="c", subcore_axis_name="s", num_cores=NC, num_subcores=NS)

_agg_out_type = jax.ShapeDtypeStruct((NC, N, D), jnp.float32)


@functools.partial(
    pl.kernel,
    out_type=_agg_out_type,
    mesh=_mesh,
    scratch_types=[
        pltpu.VMEM((2, IB, CH), jnp.int32),
        pltpu.VMEM((2, IB, CH), jnp.int32),
        pltpu.VMEM((2, CH, D), jnp.float32),
        pltpu.VMEM((2, CH, D), jnp.float32),
        pltpu.VMEM((ZR, D), jnp.float32),
        pltpu.SemaphoreType.DMA((2,)),
        pltpu.SemaphoreType.DMA((2,)),
        pltpu.SemaphoreType.DMA((2,)),
        pltpu.SemaphoreType.DMA((2,)),
        pltpu.VMEM_SHARED((ACC_R, D), jnp.float32),
    ],
)
def _sc_agg(h_hbm, ea_hbm, src_hbm, dst_hbm, out_hbm,
            src_v, dst_v, hbuf, ebuf, zbuf, gsh, gse, ssh, sse, acc):
    c = lax.axis_index("c")
    s = lax.axis_index("s")

    def zstore(t, carry):
        i = t // (D // 16)
        k = t % (D // 16)
        zbuf[i, pl.ds(k * 16, 16)] = jnp.zeros((16,), jnp.float32)
        return carry
    lax.fori_loop(0, ZR * (D // 16), zstore, 0)
    for kk in range(RPS // ZR):
        pltpu.sync_copy(zbuf, acc.at[pl.ds(s * RPS + kk * ZR, ZR)])

    t = c * NS + s
    plsc.subcore_barrier()
    base = t * EPT

    def issue_gathers(j, b, gb):
        jj = j - (j // IB) * IB
        pltpu.async_copy(h_hbm.at[src_v.at[gb, jj]], hbuf.at[b], gsh.at[b])
        pltpu.async_copy(ea_hbm.at[pl.ds(base + j * CH, CH)], ebuf.at[b],
                         gse.at[b])

    pltpu.sync_copy(src_hbm.at[t, 0], src_v.at[0])
    pltpu.sync_copy(dst_hbm.at[t, 0], dst_v.at[0])
    issue_gathers(0, 0, 0)

    def chunk(j, carry):
        g = j // IB
        jj = j - g * IB
        b = lax.rem(j, 2)
        gb = lax.rem(g, 2)
        pltpu.make_async_copy(h_hbm.at[src_v.at[gb, jj]], hbuf.at[b],
                              gsh.at[b]).wait()
        pltpu.make_async_copy(ea_hbm.at[pl.ds(base + j * CH, CH)],
                              ebuf.at[b], gse.at[b]).wait()
        pltpu.async_copy(hbuf.at[b], acc.at[dst_v.at[gb, jj]], ssh.at[b],
                         add=True)
        pltpu.async_copy(ebuf.at[b], acc.at[dst_v.at[gb, jj]], sse.at[b],
                         add=True)

        @pl.when(j + 1 < NCHUNK)
        def _():
            j1 = j + 1
            g1 = j1 // IB
            jj1 = j1 - g1 * IB
            nb = lax.rem(j1, 2)
            g1b = lax.rem(g1, 2)

            @pl.when(jj1 == 0)
            def _():
                pltpu.sync_copy(src_hbm.at[t, g1], src_v.at[g1b])
                pltpu.sync_copy(dst_hbm.at[t, g1], dst_v.at[g1b])

            @pl.when(j1 >= 2)
            def _():
                pltpu.make_async_copy(hbuf.at[nb], acc.at[dst_v.at[g1b, jj1]],
                                      ssh.at[nb]).wait()
                pltpu.make_async_copy(ebuf.at[nb], acc.at[dst_v.at[g1b, jj1]],
                                      sse.at[nb]).wait()
            issue_gathers(j1, nb, g1b)
        return carry
    lax.fori_loop(0, NCHUNK, chunk, 0)
    for b in range(2):
        pltpu.make_async_copy(hbuf.at[b], acc.at[dst_v.at[0, 0]],
                              ssh.at[b]).wait()
        pltpu.make_async_copy(ebuf.at[b], acc.at[dst_v.at[0, 0]],
                              sse.at[b]).wait()
    plsc.subcore_barrier()
    w = jnp.where(s == NS - 1, N - WPS, s * WPS)
    pltpu.sync_copy(acc.at[pl.ds(w, WPS)], out_hbm.at[c, pl.ds(w, WPS)])


def _gelu(x):
    return 0.5 * x * (1.0 + lax.erf(x * (1.0 / math.sqrt(2.0))))


def _ln(x, g, b):
    mu = jnp.mean(x, axis=-1, keepdims=True)
    var = jnp.mean((x - mu) ** 2, axis=-1, keepdims=True)
    return (x - mu) * lax.rsqrt(var + 1e-5) * g + b


def _mlp_body(a, h_ref, w1, b1, g1, bb1, w2, b2, g2, bb2, out_ref):
    h = h_ref[...]
    x = a.at[0][...] + a.at[1][...] + h
    u = jnp.dot(x, w1[...], preferred_element_type=jnp.float32) + b1[...]
    u = _gelu(_ln(u, g1[...], bb1[...]))
    v = jnp.dot(u, w2[...], preferred_element_type=jnp.float32) + b2[...]
    v = _ln(v, g2[...], bb2[...])
    out_ref[...] = _gelu(v + h)


_ROWS = 1000


def _tc_mlp(a, h, w1, b1, g1, bb1, w2, b2, g2, bb2):
    aspec = pl.BlockSpec((NC, _ROWS, D), lambda i: (0, i, 0))
    big = pl.BlockSpec((_ROWS, D), lambda i: (i, 0))
    wspec = pl.BlockSpec((D, D), lambda i: (0, 0))
    vspec = pl.BlockSpec((1, D), lambda i: (0, 0))
    return pl.pallas_call(
        _mlp_body,
        grid=(N // _ROWS,),
        in_specs=[aspec, big,
                  wspec, vspec, vspec, vspec,
                  wspec, vspec, vspec, vspec],
        out_specs=big,
        out_shape=jax.ShapeDtypeStruct((N, D), jnp.float32),
    )(a, h, w1, b1, g1, bb1, w2, b2, g2, bb2)


def kernel(h, batch, edge_index, h_edge_attr,
           W1, b1, ln1_g, ln1_b, W2, b2, ln2_g, ln2_b):
    del batch
    src = edge_index[0].astype(jnp.int32).reshape(TILES, NG, IB, CH)
    dst = edge_index[1].astype(jnp.int32).reshape(TILES, NG, IB, CH)

    def body(x, ws):
        w1, bb1v, g1, bv1, w2, bb2v, g2, bv2 = ws
        a = _sc_agg(x, h_edge_attr, src, dst)
        x = _tc_mlp(a, x, w1, bb1v, g1, bv1, w2, bb2v, g2, bv2)
        return x, None

    ws = (W1, b1.reshape(L, 1, D), ln1_g.reshape(L, 1, D),
          ln1_b.reshape(L, 1, D), W2, b2.reshape(L, 1, D),
          ln2_g.reshape(L, 1, D), ln2_b.reshape(L, 1, D))
    x, _ = lax.scan(body, h, ws)
    return x


# speedup vs baseline: 6.8281x; 1.2373x over previous
import functools
import math

import jax
import jax.numpy as jnp
from jax import lax
from jax.experimental import pallas as pl
from jax.experimental.pallas import tpu as pltpu
from jax.experimental.pallas import tpu_sc as plsc

N = 10000
E = 320000
D = 128
L = 3

NC = 2
NS = 16
TILES = NC * NS
EPT = E // TILES
CH = 80
NCHUNK = EPT // CH
IB = 25
NG = NCHUNK // IB
ACC_R = 10240
RPS = ACC_R // NS
WPS = 640
ZR = 16

_mesh = plsc.VectorSubcoreMesh(
    core_axis_name="c", subcore_axis_name="s", num_cores=NC, num_subcores=NS)

_agg_out_type = jax.ShapeDtypeStruct((NC, N, D), jnp.float32)


@functools.partial(
    pl.kernel,
    out_type=_agg_out_type,
    mesh=_mesh,
    scratch_types=[
        pltpu.VMEM((2, IB, CH), jnp.int32),
        pltpu.VMEM((2, IB, CH), jnp.int32),
        pltpu.VMEM((2, CH, D), jnp.float32),
        pltpu.VMEM((ZR, D), jnp.float32),
        pltpu.SemaphoreType.DMA((2,)),
        pltpu.SemaphoreType.DMA((2,)),
        pltpu.VMEM_SHARED((ACC_R, D), jnp.float32),
    ],
)
def _sc_agg(tab_hbm, src_hbm, dst_hbm, out_hbm,
            src_v, dst_v, rbuf, zbuf, gsem, ssem, acc):
    c = lax.axis_index("c")
    s = lax.axis_index("s")

    def zstore(t, carry):
        i = t // (D // 16)
        k = t % (D // 16)
        zbuf[i, pl.ds(k * 16, 16)] = jnp.zeros((16,), jnp.float32)
        return carry
    lax.fori_loop(0, ZR * (D // 16), zstore, 0)
    for kk in range(RPS // ZR):
        pltpu.sync_copy(zbuf, acc.at[pl.ds(s * RPS + kk * ZR, ZR)])

    t = c * NS + s
    plsc.subcore_barrier()

    def issue_gather(j, b, gb):
        jj = j - (j // IB) * IB
        pltpu.async_copy(tab_hbm.at[src_v.at[gb, jj]], rbuf.at[b],
                         gsem.at[b])

    pltpu.sync_copy(src_hbm.at[t, 0], src_v.at[0])
    pltpu.sync_copy(dst_hbm.at[t, 0], dst_v.at[0])
    issue_gather(0, 0, 0)

    def chunk(j, carry):
        g = j // IB
        jj = j - g * IB
        b = lax.rem(j, 2)
        gb = lax.rem(g, 2)
        pltpu.make_async_copy(tab_hbm.at[src_v.at[gb, jj]], rbuf.at[b],
                              gsem.at[b]).wait()
        pltpu.async_copy(rbuf.at[b], acc.at[dst_v.at[gb, jj]], ssem.at[b],
                         add=True)

        @pl.when(j + 1 < NCHUNK)
        def _():
            j1 = j + 1
            g1 = j1 // IB
            jj1 = j1 - g1 * IB
            nb = lax.rem(j1, 2)
            g1b = lax.rem(g1, 2)

            @pl.when(jj1 == 0)
            def _():
                pltpu.sync_copy(src_hbm.at[t, g1], src_v.at[g1b])
                pltpu.sync_copy(dst_hbm.at[t, g1], dst_v.at[g1b])

            @pl.when(j1 >= 2)
            def _():
                pltpu.make_async_copy(rbuf.at[nb], acc.at[dst_v.at[g1b, jj1]],
                                      ssem.at[nb]).wait()
            issue_gather(j1, nb, g1b)
        return carry
    lax.fori_loop(0, NCHUNK, chunk, 0)
    for b in range(2):
        pltpu.make_async_copy(rbuf.at[b], acc.at[dst_v.at[0, 0]],
                              ssem.at[b]).wait()
    plsc.subcore_barrier()
    w = jnp.where(s == NS - 1, N - WPS, s * WPS)
    pltpu.sync_copy(acc.at[pl.ds(w, WPS)], out_hbm.at[c, pl.ds(w, WPS)])


def _gelu(x):
    return 0.5 * x * (1.0 + lax.erf(x * (1.0 / math.sqrt(2.0))))


def _ln(x, g, b):
    mu = jnp.mean(x, axis=-1, keepdims=True)
    var = jnp.mean((x - mu) ** 2, axis=-1, keepdims=True)
    return (x - mu) * lax.rsqrt(var + 1e-5) * g + b


def _mlp_body(a, e, h_ref, w1, b1, g1, bb1, w2, b2, g2, bb2, out_ref):
    h = h_ref[...]
    x = a.at[0][...] + a.at[1][...] + e.at[0][...] + e.at[1][...] + h
    u = jnp.dot(x, w1[...], preferred_element_type=jnp.float32) + b1[...]
    u = _gelu(_ln(u, g1[...], bb1[...]))
    v = jnp.dot(u, w2[...], preferred_element_type=jnp.float32) + b2[...]
    v = _ln(v, g2[...], bb2[...])
    out_ref[...] = _gelu(v + h)


_ROWS = 1000


def _tc_mlp(a, e, x_pad, w1, b1, g1, bb1, w2, b2, g2, bb2):
    aspec = pl.BlockSpec((NC, _ROWS, D), lambda i: (0, i, 0))
    big = pl.BlockSpec((_ROWS, D), lambda i: (i, 0))
    wspec = pl.BlockSpec((D, D), lambda i: (0, 0))
    vspec = pl.BlockSpec((1, D), lambda i: (0, 0))
    return pl.pallas_call(
        _mlp_body,
        grid=(N // _ROWS,),
        in_specs=[aspec, aspec, big,
                  wspec, vspec, vspec, vspec,
                  wspec, vspec, vspec, vspec],
        out_specs=big,
        out_shape=jax.ShapeDtypeStruct((E, D), jnp.float32),
        input_output_aliases={2: 0},
    )(a, e, x_pad, w1, b1, g1, bb1, w2, b2, g2, bb2)


def kernel(h, batch, edge_index, h_edge_attr,
           W1, b1, ln1_g, ln1_b, W2, b2, ln2_g, ln2_b):
    del batch
    src = edge_index[0].astype(jnp.int32).reshape(TILES, NG, IB, CH)
    dst = edge_index[1].astype(jnp.int32).reshape(TILES, NG, IB, CH)
    eidx = jnp.arange(E, dtype=jnp.int32).reshape(TILES, NG, IB, CH)

    e = _sc_agg(h_edge_attr, eidx, dst)

    x_pad = jnp.concatenate(
        [h, jnp.zeros((E - N, D), dtype=h.dtype)], axis=0)

    def body(x_pad, ws):
        w1, bb1v, g1, bv1, w2, bb2v, g2, bv2 = ws
        a = _sc_agg(x_pad, src, dst)
        x_pad = _tc_mlp(a, e, x_pad, w1, bb1v, g1, bv1, w2, bb2v, g2, bv2)
        return x_pad, None

    ws = (W1, b1.reshape(L, 1, D), ln1_g.reshape(L, 1, D),
          ln1_b.reshape(L, 1, D), W2, b2.reshape(L, 1, D),
          ln2_g.reshape(L, 1, D), ln2_b.reshape(L, 1, D))
    x_pad, _ = lax.scan(body, x_pad, ws)
    return x_pad[:N]


# speedup vs baseline: 7.0414x; 1.0312x over previous
import functools
import math

import jax
import jax.numpy as jnp
from jax import lax
from jax.experimental import pallas as pl
from jax.experimental.pallas import tpu as pltpu
from jax.experimental.pallas import tpu_sc as plsc

N = 10000
E = 320000
D = 128
L = 3

NC = 2
NS = 16
TILES = NC * NS
EPT = E // TILES
CH = 80
NCHUNK = EPT // CH
IB = 25
NG = NCHUNK // IB
ACC_R = 10240
RPS = ACC_R // NS
WPS = 640
ZR = 16

_mesh = plsc.VectorSubcoreMesh(
    core_axis_name="c", subcore_axis_name="s", num_cores=NC, num_subcores=NS)

_agg_out_type = jax.ShapeDtypeStruct((NC, N, D), jnp.float32)


def _zero_acc(acc, zbuf, s):
    def zstore(t, carry):
        i = t // (D // 16)
        k = t % (D // 16)
        zbuf[i, pl.ds(k * 16, 16)] = jnp.zeros((16,), jnp.float32)
        return carry
    lax.fori_loop(0, ZR * (D // 16), zstore, 0)
    for kk in range(RPS // ZR):
        pltpu.sync_copy(zbuf, acc.at[pl.ds(s * RPS + kk * ZR, ZR)])


def _write_out(acc, out_hbm, c, s):
    w = jnp.where(s == NS - 1, N - WPS, s * WPS)
    pltpu.sync_copy(acc.at[pl.ds(w, WPS)], out_hbm.at[c, pl.ds(w, WPS)])


@functools.partial(
    pl.kernel,
    out_type=_agg_out_type,
    mesh=_mesh,
    scratch_types=[
        pltpu.VMEM((2, IB, CH), jnp.int32),
        pltpu.VMEM((2, IB, CH), jnp.int32),
        pltpu.VMEM((2, CH, D), jnp.float32),
        pltpu.VMEM((ZR, D), jnp.float32),
        pltpu.SemaphoreType.DMA((2,)),
        pltpu.SemaphoreType.DMA((2,)),
        pltpu.VMEM_SHARED((ACC_R, D), jnp.float32),
    ],
)
def _sc_agg(tab_hbm, src_hbm, dst_hbm, out_hbm,
            src_v, dst_v, rbuf, zbuf, gsem, ssem, acc):
    c = lax.axis_index("c")
    s = lax.axis_index("s")
    _zero_acc(acc, zbuf, s)
    t = c * NS + s
    plsc.subcore_barrier()

    def issue_gather(j, b, gb):
        jj = j - (j // IB) * IB
        pltpu.async_copy(tab_hbm.at[src_v.at[gb, jj]], rbuf.at[b],
                         gsem.at[b])

    pltpu.sync_copy(src_hbm.at[t, 0], src_v.at[0])
    pltpu.sync_copy(dst_hbm.at[t, 0], dst_v.at[0])
    issue_gather(0, 0, 0)

    def chunk(j, carry):
        g = j // IB
        jj = j - g * IB
        b = lax.rem(j, 2)
        gb = lax.rem(g, 2)
        pltpu.make_async_copy(tab_hbm.at[src_v.at[gb, jj]], rbuf.at[b],
                              gsem.at[b]).wait()
        pltpu.async_copy(rbuf.at[b], acc.at[dst_v.at[gb, jj]], ssem.at[b],
                         add=True)

        @pl.when(j + 1 < NCHUNK)
        def _():
            j1 = j + 1
            g1 = j1 // IB
            jj1 = j1 - g1 * IB
            nb = lax.rem(j1, 2)
            g1b = lax.rem(g1, 2)

            @pl.when(jj1 == 0)
            def _():
                pltpu.sync_copy(src_hbm.at[t, g1], src_v.at[g1b])
                pltpu.sync_copy(dst_hbm.at[t, g1], dst_v.at[g1b])

            @pl.when(j1 >= 2)
            def _():
                pltpu.make_async_copy(rbuf.at[nb], acc.at[dst_v.at[g1b, jj1]],
                                      ssem.at[nb]).wait()
            issue_gather(j1, nb, g1b)
        return carry
    lax.fori_loop(0, NCHUNK, chunk, 0)
    for b in range(2):
        pltpu.make_async_copy(rbuf.at[b], acc.at[dst_v.at[0, 0]],
                              ssem.at[b]).wait()
    plsc.subcore_barrier()
    _write_out(acc, out_hbm, c, s)


@functools.partial(
    pl.kernel,
    out_type=_agg_out_type,
    mesh=_mesh,
    scratch_types=[
        pltpu.VMEM((2, IB, CH), jnp.int32),
        pltpu.VMEM((2, CH, D), jnp.float32),
        pltpu.VMEM((ZR, D), jnp.float32),
        pltpu.SemaphoreType.DMA((2,)),
        pltpu.SemaphoreType.DMA((2,)),
        pltpu.VMEM_SHARED((ACC_R, D), jnp.float32),
    ],
)
def _sc_agg_linear(ea_hbm, dst_hbm, out_hbm,
                   dst_v, rbuf, zbuf, gsem, ssem, acc):
    c = lax.axis_index("c")
    s = lax.axis_index("s")
    _zero_acc(acc, zbuf, s)
    t = c * NS + s
    plsc.subcore_barrier()
    base = t * EPT

    def issue_stream(j, b):
        pltpu.async_copy(ea_hbm.at[pl.ds(base + j * CH, CH)], rbuf.at[b],
                         gsem.at[b])

    pltpu.sync_copy(dst_hbm.at[t, 0], dst_v.at[0])
    issue_stream(0, 0)

    def chunk(j, carry):
        g = j // IB
        jj = j - g * IB
        b = lax.rem(j, 2)
        gb = lax.rem(g, 2)
        pltpu.make_async_copy(ea_hbm.at[pl.ds(base + j * CH, CH)],
                              rbuf.at[b], gsem.at[b]).wait()
        pltpu.async_copy(rbuf.at[b], acc.at[dst_v.at[gb, jj]], ssem.at[b],
                         add=True)

        @pl.when(j + 1 < NCHUNK)
        def _():
            j1 = j + 1
            g1 = j1 // IB
            jj1 = j1 - g1 * IB
            nb = lax.rem(j1, 2)
            g1b = lax.rem(g1, 2)

            @pl.when(jj1 == 0)
            def _():
                pltpu.sync_copy(dst_hbm.at[t, g1], dst_v.at[g1b])

            @pl.when(j1 >= 2)
            def _():
                pltpu.make_async_copy(rbuf.at[nb], acc.at[dst_v.at[g1b, jj1]],
                                      ssem.at[nb]).wait()
            issue_stream(j1, nb)
        return carry
    lax.fori_loop(0, NCHUNK, chunk, 0)
    for b in range(2):
        pltpu.make_async_copy(rbuf.at[b], acc.at[dst_v.at[0, 0]],
                              ssem.at[b]).wait()
    plsc.subcore_barrier()
    _write_out(acc, out_hbm, c, s)


def _gelu(x):
    return 0.5 * x * (1.0 + lax.erf(x * (1.0 / math.sqrt(2.0))))


def _ln(x, g, b):
    mu = jnp.mean(x, axis=-1, keepdims=True)
    var = jnp.mean((x - mu) ** 2, axis=-1, keepdims=True)
    return (x - mu) * lax.rsqrt(var + 1e-5) * g + b


def _mlp_body(a, e, h_ref, w1, b1, g1, bb1, w2, b2, g2, bb2, out_ref):
    h = h_ref[...]
    x = a.at[0][...] + a.at[1][...] + e.at[0][...] + e.at[1][...] + h
    u = jnp.dot(x, w1[...], preferred_element_type=jnp.float32) + b1[...]
    u = _gelu(_ln(u, g1[...], bb1[...]))
    v = jnp.dot(u, w2[...], preferred_element_type=jnp.float32) + b2[...]
    v = _ln(v, g2[...], bb2[...])
    out_ref[...] = _gelu(v + h)


_ROWS = 1000


def _tc_mlp(a, e, h, w1, b1, g1, bb1, w2, b2, g2, bb2):
    aspec = pl.BlockSpec((NC, _ROWS, D), lambda i: (0, i, 0))
    big = pl.BlockSpec((_ROWS, D), lambda i: (i, 0))
    wspec = pl.BlockSpec((D, D), lambda i: (0, 0))
    vspec = pl.BlockSpec((1, D), lambda i: (0, 0))
    return pl.pallas_call(
        _mlp_body,
        grid=(N // _ROWS,),
        in_specs=[aspec, aspec, big,
                  wspec, vspec, vspec, vspec,
                  wspec, vspec, vspec, vspec],
        out_specs=big,
        out_shape=jax.ShapeDtypeStruct((N, D), jnp.float32),
    )(a, e, h, w1, b1, g1, bb1, w2, b2, g2, bb2)


def kernel(h, batch, edge_index, h_edge_attr,
           W1, b1, ln1_g, ln1_b, W2, b2, ln2_g, ln2_b):
    del batch
    src = edge_index[0].astype(jnp.int32).reshape(TILES, NG, IB, CH)
    dst = edge_index[1].astype(jnp.int32).reshape(TILES, NG, IB, CH)

    e = _sc_agg_linear(h_edge_attr, dst)

    def body(x, ws):
        w1, bb1v, g1, bv1, w2, bb2v, g2, bv2 = ws
        a = _sc_agg(x, src, dst)
        x = _tc_mlp(a, e, x, w1, bb1v, g1, bv1, w2, bb2v, g2, bv2)
        return x, None

    ws = (W1, b1.reshape(L, 1, D), ln1_g.reshape(L, 1, D),
          ln1_b.reshape(L, 1, D), W2, b2.reshape(L, 1, D),
          ln2_g.reshape(L, 1, D), ln2_b.reshape(L, 1, D))
    x, _ = lax.scan(body, h, ws)
    return x


# speedup vs baseline: 10.0933x; 1.4334x over previous
import functools
import math

import jax
import jax.numpy as jnp
from jax import lax
from jax.experimental import pallas as pl
from jax.experimental.pallas import tpu as pltpu
from jax.experimental.pallas import tpu_sc as plsc

N = 10000
E = 320000
D = 128
L = 3

NC = 2
NS = 16
TILES = NC * NS
EPT = E // TILES
CH = 40
NCHUNK = EPT // CH
IB = 25
NG = NCHUNK // IB
K = 6
ACC_R = N
WPS = 640
ZR = 16

_mesh = plsc.VectorSubcoreMesh(
    core_axis_name="c", subcore_axis_name="s", num_cores=NC, num_subcores=NS)

_agg_out_type = jax.ShapeDtypeStruct((NC, N, D), jnp.float32)


def _zero_acc(acc, zbuf, s):
    def zstore(t, carry):
        i = t // (D // 16)
        k = t % (D // 16)
        zbuf[i, pl.ds(k * 16, 16)] = jnp.zeros((16,), jnp.float32)
        return carry
    lax.fori_loop(0, ZR * (D // 16), zstore, 0)
    z = jnp.where(s == NS - 1, N - WPS, s * WPS)
    for kk in range(WPS // ZR):
        pltpu.sync_copy(zbuf, acc.at[pl.ds(z + kk * ZR, ZR)])


def _write_out(acc, out_hbm, c, s):
    w = jnp.where(s == NS - 1, N - WPS, s * WPS)
    pltpu.sync_copy(acc.at[pl.ds(w, WPS)], out_hbm.at[c, pl.ds(w, WPS)])


def _pipelined_agg(issue_fetch, wait_fetch, reload_idx,
                   dst_v, rbuf, ssem, acc):
    reload_idx(0, 0)
    for j in range(K - 1):
        issue_fetch(j, j % K)

    def chunk(j, carry):
        g = j // IB
        jj = j - g * IB
        b = lax.rem(j, K)
        gb = lax.rem(g, 2)
        wait_fetch(j, b)
        pltpu.async_copy(rbuf.at[b], acc.at[dst_v.at[gb, jj]], ssem.at[b],
                         add=True)

        @pl.when(j + K - 1 < NCHUNK)
        def _():
            j1 = j + K - 1
            g1 = j1 // IB
            jj1 = j1 - g1 * IB
            nb = lax.rem(j1, K)
            g1b = lax.rem(g1, 2)

            @pl.when(jj1 == 0)
            def _():
                reload_idx(g1, g1b)

            @pl.when(j1 >= K)
            def _():
                pltpu.make_async_copy(rbuf.at[nb], acc.at[dst_v.at[g1b, jj1]],
                                      ssem.at[nb]).wait()
            issue_fetch(j1, nb)
        return carry
    lax.fori_loop(0, NCHUNK, chunk, 0)
    for r in range(K - 1):
        b = (NCHUNK - K + 1 + r) % K
        pltpu.make_async_copy(rbuf.at[b], acc.at[dst_v.at[0, 0]],
                              ssem.at[b]).wait()


@functools.partial(
    pl.kernel,
    out_type=_agg_out_type,
    mesh=_mesh,
    scratch_types=[
        pltpu.VMEM((2, IB, CH), jnp.int32),
        pltpu.VMEM((2, IB, CH), jnp.int32),
        pltpu.VMEM((K, CH, D), jnp.float32),
        pltpu.VMEM((ZR, D), jnp.float32),
        pltpu.SemaphoreType.DMA((K,)),
        pltpu.SemaphoreType.DMA((K,)),
        pltpu.VMEM_SHARED((ACC_R, D), jnp.float32),
    ],
)
def _sc_agg(tab_hbm, src_hbm, dst_hbm, out_hbm,
            src_v, dst_v, rbuf, zbuf, gsem, ssem, acc):
    c = lax.axis_index("c")
    s = lax.axis_index("s")
    _zero_acc(acc, zbuf, s)
    t = c * NS + s
    plsc.subcore_barrier()

    def issue_fetch(j, b):
        g = j // IB
        jj = j - g * IB
        gb = lax.rem(g, 2) if not isinstance(g, int) else g % 2
        pltpu.async_copy(tab_hbm.at[src_v.at[gb, jj]], rbuf.at[b],
                         gsem.at[b])

    def wait_fetch(j, b):
        g = j // IB
        jj = j - g * IB
        gb = lax.rem(g, 2) if not isinstance(g, int) else g % 2
        pltpu.make_async_copy(tab_hbm.at[src_v.at[gb, jj]], rbuf.at[b],
                              gsem.at[b]).wait()

    def reload_idx(g1, g1b):
        pltpu.sync_copy(src_hbm.at[t, g1], src_v.at[g1b])
        pltpu.sync_copy(dst_hbm.at[t, g1], dst_v.at[g1b])

    _pipelined_agg(issue_fetch, wait_fetch, reload_idx, dst_v, rbuf, ssem,
                   acc)
    plsc.subcore_barrier()
    _write_out(acc, out_hbm, c, s)


@functools.partial(
    pl.kernel,
    out_type=_agg_out_type,
    mesh=_mesh,
    scratch_types=[
        pltpu.VMEM((2, IB, CH), jnp.int32),
        pltpu.VMEM((K, CH, D), jnp.float32),
        pltpu.VMEM((ZR, D), jnp.float32),
        pltpu.SemaphoreType.DMA((K,)),
        pltpu.SemaphoreType.DMA((K,)),
        pltpu.VMEM_SHARED((ACC_R, D), jnp.float32),
    ],
)
def _sc_agg_linear(ea_hbm, dst_hbm, out_hbm,
                   dst_v, rbuf, zbuf, gsem, ssem, acc):
    c = lax.axis_index("c")
    s = lax.axis_index("s")
    _zero_acc(acc, zbuf, s)
    t = c * NS + s
    plsc.subcore_barrier()
    base = t * EPT

    def issue_fetch(j, b):
        pltpu.async_copy(ea_hbm.at[pl.ds(base + j * CH, CH)], rbuf.at[b],
                         gsem.at[b])

    def wait_fetch(j, b):
        pltpu.make_async_copy(ea_hbm.at[pl.ds(base + j * CH, CH)],
                              rbuf.at[b], gsem.at[b]).wait()

    def reload_idx(g1, g1b):
        pltpu.sync_copy(dst_hbm.at[t, g1], dst_v.at[g1b])

    _pipelined_agg(issue_fetch, wait_fetch, reload_idx, dst_v, rbuf, ssem,
                   acc)
    plsc.subcore_barrier()
    _write_out(acc, out_hbm, c, s)


def _gelu(x):
    return 0.5 * x * (1.0 + lax.erf(x * (1.0 / math.sqrt(2.0))))


def _ln(x, g, b):
    mu = jnp.mean(x, axis=-1, keepdims=True)
    var = jnp.mean((x - mu) ** 2, axis=-1, keepdims=True)
    return (x - mu) * lax.rsqrt(var + 1e-5) * g + b


def _mlp_body(a, e, h_ref, w1, b1, g1, bb1, w2, b2, g2, bb2, out_ref):
    h = h_ref[...]
    x = a.at[0][...] + a.at[1][...] + e.at[0][...] + e.at[1][...] + h
    u = jnp.dot(x, w1[...], preferred_element_type=jnp.float32) + b1[...]
    u = _gelu(_ln(u, g1[...], bb1[...]))
    v = jnp.dot(u, w2[...], preferred_element_type=jnp.float32) + b2[...]
    v = _ln(v, g2[...], bb2[...])
    out_ref[...] = _gelu(v + h)


_ROWS = 1000


def _tc_mlp(a, e, h, w1, b1, g1, bb1, w2, b2, g2, bb2):
    aspec = pl.BlockSpec((NC, _ROWS, D), lambda i: (0, i, 0))
    big = pl.BlockSpec((_ROWS, D), lambda i: (i, 0))
    wspec = pl.BlockSpec((D, D), lambda i: (0, 0))
    vspec = pl.BlockSpec((1, D), lambda i: (0, 0))
    return pl.pallas_call(
        _mlp_body,
        grid=(N // _ROWS,),
        in_specs=[aspec, aspec, big,
                  wspec, vspec, vspec, vspec,
                  wspec, vspec, vspec, vspec],
        out_specs=big,
        out_shape=jax.ShapeDtypeStruct((N, D), jnp.float32),
    )(a, e, h, w1, b1, g1, bb1, w2, b2, g2, bb2)


def kernel(h, batch, edge_index, h_edge_attr,
           W1, b1, ln1_g, ln1_b, W2, b2, ln2_g, ln2_b):
    del batch
    src = edge_index[0].astype(jnp.int32).reshape(TILES, NG, IB, CH)
    dst = edge_index[1].astype(jnp.int32).reshape(TILES, NG, IB, CH)

    e = _sc_agg_linear(h_edge_attr, dst)

    def body(x, ws):
        w1, bb1v, g1, bv1, w2, bb2v, g2, bv2 = ws
        a = _sc_agg(x, src, dst)
        x = _tc_mlp(a, e, x, w1, bb1v, g1, bv1, w2, bb2v, g2, bv2)
        return x, None

    ws = (W1, b1.reshape(L, 1, D), ln1_g.reshape(L, 1, D),
          ln1_b.reshape(L, 1, D), W2, b2.reshape(L, 1, D),
          ln2_g.reshape(L, 1, D), ln2_b.reshape(L, 1, D))
    x, _ = lax.scan(body, h, ws)
    return x
